# Initial kernel scaffold; baseline (speedup 1.0000x reference)
import jax
import jax.numpy as jnp
from jax.experimental import pallas as pl


def kernel(x_seq, edge_index, W1, a_src1, a_dst1, b1, W2, a_src2, a_dst2, b2, P2, Wih0, Whh0, bih0, bhh0, Wih1, Whh1, bih1, bhh1, Wfh, bfh, Wfo, bfo):
    raise NotImplementedError("write your pallas kernel here")


# speedup vs baseline: 19.2286x; 19.2286x over previous
import functools

import jax
import jax.numpy as jnp
from jax import lax
from jax.experimental import pallas as pl
from jax.experimental.pallas import tpu as pltpu
from jax.experimental.pallas import tpu_sc as plsc

N = 10000
N_PAD = 10240
T = 4
IN_CH = 128
GAT_C = 32
GRU_H = 32

NW = 32
NCH = 42
K = 128
E_PAD = NW * NCH * K

R1 = 144
SB1, EB1 = 128, 136
R2 = 64
SB2, EB2 = 32, 40

ROWS_PER_TILE = N_PAD // 16
ZCP = ROWS_PER_TILE // K


@functools.lru_cache(maxsize=None)
def _make_sc_edge_agg(R, H, SB, EB):
    mesh = plsc.VectorSubcoreMesh(core_axis_name="c", subcore_axis_name="s")

    @functools.partial(
        pl.kernel,
        out_type=jax.ShapeDtypeStruct((2, N_PAD, R), jnp.float32),
        mesh=mesh,
        compiler_params=pltpu.CompilerParams(
            use_tc_tiling_on_sc=False, needs_layout_passes=False),
        scratch_types=[
            pltpu.VMEM((NCH, K), jnp.int32),
            pltpu.VMEM((NCH, K), jnp.int32),
            pltpu.VMEM((K, R), jnp.float32),
            pltpu.VMEM((K, 16), jnp.float32),
            pltpu.VMEM_SHARED((N_PAD, R), jnp.float32),
            pltpu.SemaphoreType.DMA,
            pltpu.SemaphoreType.DMA,
        ],
    )
    def sc_kernel(xwc_hbm, ed_hbm, src_hbm, dst_hbm, tok_hbm, out_hbm,
                  src_w, dst_w, rows, de, acc, sem1, sem2):
        del tok_hbm
        cid = lax.axis_index("c")
        sid = lax.axis_index("s")
        wid = sid * 2 + cid
        lanes0 = lax.iota(jnp.int32, 16)
        zv = jnp.zeros((16,), jnp.float32)

        pltpu.sync_copy(src_hbm.at[wid], src_w)
        pltpu.sync_copy(dst_hbm.at[wid], dst_w)

        def zrow(r, c):
            def zcol(k, c2):
                plsc.store_scatter(
                    rows, [jnp.full((16,), r, jnp.int32), k * 16 + lanes0], zv)
                return c2
            return lax.fori_loop(0, R // 16, zcol, c)
        lax.fori_loop(0, K, zrow, 0)

        base_n = sid * ROWS_PER_TILE

        def zcp(i, c):
            pltpu.sync_copy(rows, acc.at[pl.ds(base_n + i * K, K)])
            return c
        lax.fori_loop(0, ZCP, zcp, 0)
        plsc.subcore_barrier()

        def chunk(j, c):
            g1 = pltpu.async_copy(xwc_hbm.at[src_w.at[j]], rows, sem1)
            g2 = pltpu.async_copy(ed_hbm.at[dst_w.at[j]], de, sem2)
            g1.wait()
            g2.wait()

            def grp(g, c2):
                lanes = g * 16 + lanes0
                for h in range(H):
                    s = plsc.load_gather(
                        rows, [lanes, jnp.full((16,), SB + h, jnp.int32)])
                    d = plsc.load_gather(
                        de, [lanes, jnp.full((16,), h, jnp.int32)])
                    a = s + d
                    a = jnp.maximum(a, 0.2 * a)
                    e = jnp.exp(a)
                    plsc.store_scatter(
                        rows, [lanes, jnp.full((16,), EB + h, jnp.int32)], e)

                    def col(cc, c3):
                        colv = jnp.full((16,), 0, jnp.int32) + cc
                        v = plsc.load_gather(rows, [lanes, colv]) * e
                        plsc.store_scatter(rows, [lanes, colv], v)
                        return c3
                    lax.fori_loop(h * 32, h * 32 + 32, col, c2)
                return c2
            lax.fori_loop(0, K // 16, grp, 0)

            pltpu.sync_copy(rows, acc.at[dst_w.at[j]], add=True)
            return c
        lax.fori_loop(0, NCH, chunk, 0)
        plsc.subcore_barrier()

        def wout(i, c):
            pltpu.sync_copy(acc.at[pl.ds(base_n + i * K, K)],
                            out_hbm.at[cid, pl.ds(base_n + i * K, K)])
            return c
        lax.fori_loop(0, ZCP, wout, 0)

    return sc_kernel


_BLK = 512


def _tc_pre_body(x_ref, wc_ref, wd_ref, xwc_ref, ed_ref):
    x = x_ref[...]
    xwc_ref[...] = jnp.dot(x, wc_ref[...], preferred_element_type=jnp.float32)
    ed_ref[...] = jnp.dot(x, wd_ref[...], preferred_element_type=jnp.float32)


def _tc_pre(x_all, wcomb1, wd1):
    g = x_all.shape[0] // _BLK
    return pl.pallas_call(
        _tc_pre_body,
        grid=(g,),
        in_specs=[
            pl.BlockSpec((_BLK, IN_CH), lambda i: (i, 0)),
            pl.BlockSpec((IN_CH, R1), lambda i: (0, 0)),
            pl.BlockSpec((IN_CH, 16), lambda i: (0, 0)),
        ],
        out_specs=[
            pl.BlockSpec((_BLK, R1), lambda i: (i, 0)),
            pl.BlockSpec((_BLK, 16), lambda i: (i, 0)),
        ],
        out_shape=[
            jax.ShapeDtypeStruct((x_all.shape[0], R1), jnp.float32),
            jax.ShapeDtypeStruct((x_all.shape[0], 16), jnp.float32),
        ],
    )(x_all, wcomb1, wd1)


def _tc_mid_body(p_ref, x_ref, b1_ref, bsel_ref, wc2_ref, wd2_ref,
                 xn_ref, xwc2_ref, ed2_ref):
    num = p_ref[0, :, :IN_CH] + p_ref[1, :, :IN_CH]
    tail = p_ref[0, :, IN_CH:R1] + p_ref[1, :, IN_CH:R1]
    den = jnp.dot(tail, bsel_ref[...], preferred_element_type=jnp.float32)
    g1 = num / (den + 1e-16) + b1_ref[...]
    xn = jnp.where(g1 > 0, g1, jnp.exp(g1) - 1.0) + x_ref[...]
    xn_ref[...] = xn
    xwc2_ref[...] = jnp.dot(xn, wc2_ref[...], preferred_element_type=jnp.float32)
    ed2_ref[...] = jnp.dot(xn, wd2_ref[...], preferred_element_type=jnp.float32)


def _tc_mid(part1, x_t, b1r, bsel1, wcomb2, wd2):
    g = N_PAD // _BLK
    return pl.pallas_call(
        _tc_mid_body,
        grid=(g,),
        in_specs=[
            pl.BlockSpec((2, _BLK, R1), lambda i: (0, i, 0)),
            pl.BlockSpec((_BLK, IN_CH), lambda i: (i, 0)),
            pl.BlockSpec((1, IN_CH), lambda i: (0, 0)),
            pl.BlockSpec((R1 - IN_CH, IN_CH), lambda i: (0, 0)),
            pl.BlockSpec((IN_CH, R2), lambda i: (0, 0)),
            pl.BlockSpec((IN_CH, 16), lambda i: (0, 0)),
        ],
        out_specs=[
            pl.BlockSpec((_BLK, IN_CH), lambda i: (i, 0)),
            pl.BlockSpec((_BLK, R2), lambda i: (i, 0)),
            pl.BlockSpec((_BLK, 16), lambda i: (i, 0)),
        ],
        out_shape=[
            jax.ShapeDtypeStruct((N_PAD, IN_CH), jnp.float32),
            jax.ShapeDtypeStruct((N_PAD, R2), jnp.float32),
            jax.ShapeDtypeStruct((N_PAD, 16), jnp.float32),
        ],
    )(part1, x_t, b1r, bsel1, wcomb2, wd2)


def _gru_block(x, h, wih_t, whh_t, bih, bhh):
    gi = jnp.dot(x, wih_t, preferred_element_type=jnp.float32) + bih
    gh = jnp.dot(h, whh_t, preferred_element_type=jnp.float32) + bhh
    r = jax.nn.sigmoid(gi[:, 0:32] + gh[:, 0:32])
    z = jax.nn.sigmoid(gi[:, 32:64] + gh[:, 32:64])
    ng = jnp.tanh(gi[:, 64:96] + r * gh[:, 64:96])
    return (1.0 - z) * ng + z * h


def _tc_post_body(p_ref, xn_ref, h0_ref, h1_ref, b2_ref, bsel2_ref, p2_ref,
                  wih0_ref, whh0_ref, bih0_ref, bhh0_ref,
                  wih1_ref, whh1_ref, bih1_ref, bhh1_ref,
                  h0n_ref, h1n_ref):
    num = p_ref[0, :, :GAT_C] + p_ref[1, :, :GAT_C]
    tail = p_ref[0, :, GAT_C:R2] + p_ref[1, :, GAT_C:R2]
    den = jnp.dot(tail, bsel2_ref[...], preferred_element_type=jnp.float32)
    g2 = num / (den + 1e-16) + b2_ref[...]
    x2 = (jnp.where(g2 > 0, g2, jnp.exp(g2) - 1.0)
          + jnp.dot(xn_ref[...], p2_ref[...], preferred_element_type=jnp.float32))
    h0 = h0_ref[...]
    h1 = h1_ref[...]
    h0n = _gru_block(x2, h0, wih0_ref[...], whh0_ref[...],
                     bih0_ref[...], bhh0_ref[...])
    h1n = _gru_block(h0n, h1, wih1_ref[...], whh1_ref[...],
                     bih1_ref[...], bhh1_ref[...])
    h0n_ref[...] = h0n
    h1n_ref[...] = h1n


def _tc_post(part2, xn, h0, h1, b2r, bsel2, p2,
             wih0t, whh0t, bih0r, bhh0r, wih1t, whh1t, bih1r, bhh1r):
    g = N_PAD // _BLK
    full = lambda a, b: pl.BlockSpec((a, b), lambda i: (0, 0))
    rows = lambda b: pl.BlockSpec((_BLK, b), lambda i: (i, 0))
    return pl.pallas_call(
        _tc_post_body,
        grid=(g,),
        in_specs=[
            pl.BlockSpec((2, _BLK, R2), lambda i: (0, i, 0)),
            rows(IN_CH), rows(GRU_H), rows(GRU_H),
            full(1, GAT_C), full(R2 - GAT_C, GAT_C), full(IN_CH, GAT_C),
            full(GAT_C, 96), full(GRU_H, 96), full(1, 96), full(1, 96),
            full(GRU_H, 96), full(GRU_H, 96), full(1, 96), full(1, 96),
        ],
        out_specs=[rows(GRU_H), rows(GRU_H)],
        out_shape=[
            jax.ShapeDtypeStruct((N_PAD, GRU_H), jnp.float32),
            jax.ShapeDtypeStruct((N_PAD, GRU_H), jnp.float32),
        ],
    )(part2, xn, h0, h1, b2r, bsel2, p2,
      wih0t, whh0t, bih0r, bhh0r, wih1t, whh1t, bih1r, bhh1r)


def _tc_head_body(h1_ref, wfh_ref, bfh_ref, wfo_ref, bfo_ref, out_ref):
    y = jnp.dot(h1_ref[...], wfh_ref[...], preferred_element_type=jnp.float32)
    y = jnp.maximum(y + bfh_ref[...], 0.0)
    out_ref[...] = (jnp.dot(y, wfo_ref[...], preferred_element_type=jnp.float32)
                    + bfo_ref[...])


def _tc_head(h1, wfht, bfhr, wfo128, bfo128):
    g = N_PAD // _BLK
    full = lambda a, b: pl.BlockSpec((a, b), lambda i: (0, 0))
    return pl.pallas_call(
        _tc_head_body,
        grid=(g,),
        in_specs=[
            pl.BlockSpec((_BLK, GRU_H), lambda i: (i, 0)),
            full(GRU_H, 32), full(1, 32), full(32, 128), full(1, 128),
        ],
        out_specs=pl.BlockSpec((_BLK, 128), lambda i: (i, 0)),
        out_shape=jax.ShapeDtypeStruct((N_PAD, 128), jnp.float32),
    )(h1, wfht, bfhr, wfo128, bfo128)


def kernel(x_seq, edge_index, W1, a_src1, a_dst1, b1, W2, a_src2, a_dst2, b2,
           P2, Wih0, Whh0, bih0, bhh0, Wih1, Whh1, bih1, bhh1,
           Wfh, bfh, Wfo, bfo):
    f32 = jnp.float32

    sel_s1 = jnp.zeros((IN_CH, R1 - IN_CH), f32)
    sel_d1 = jnp.zeros((IN_CH, 16), f32)
    for h in range(4):
        blk = jnp.zeros((IN_CH,), f32).at[32 * h:32 * h + 32].set(a_src1[h])
        sel_s1 = sel_s1.at[:, h].set(blk)
        blkd = jnp.zeros((IN_CH,), f32).at[32 * h:32 * h + 32].set(a_dst1[h])
        sel_d1 = sel_d1.at[:, h].set(blkd)
    wcomb1 = jnp.concatenate([W1, W1 @ sel_s1], axis=1)
    wd1 = W1 @ sel_d1

    sel_s2 = jnp.zeros((GAT_C, R2 - GAT_C), f32).at[:, 0].set(a_src2[0])
    sel_d2 = jnp.zeros((GAT_C, 16), f32).at[:, 0].set(a_dst2[0])
    wcomb2 = jnp.concatenate([W2, W2 @ sel_s2], axis=1)
    wd2 = W2 @ sel_d2

    bsel1 = jnp.zeros((R1 - IN_CH, IN_CH), f32)
    for h in range(4):
        bsel1 = bsel1.at[EB1 - IN_CH + h, 32 * h:32 * h + 32].set(1.0)
    bsel2 = jnp.zeros((R2 - GAT_C, GAT_C), f32).at[EB2 - GAT_C, :].set(1.0)

    def pad_cols(m, w):
        return jnp.zeros((m.shape[0], w), f32).at[:, :m.shape[1]].set(m)

    wih0t, whh0t = Wih0.T, Whh0.T
    wih1t, whh1t = Wih1.T, Whh1.T
    bih0r, bhh0r = bih0.reshape(1, 96), bhh0.reshape(1, 96)
    bih1r, bhh1r = bih1.reshape(1, 96), bhh1.reshape(1, 96)
    b1r = b1.reshape(1, IN_CH)
    b2r = b2.reshape(1, GAT_C)
    wfht = Wfh.T
    bfhr = bfh.reshape(1, 32)
    wfo128 = pad_cols(Wfo.T, 128)
    bfo128 = pad_cols(bfo.reshape(1, 1), 128)

    loop = jnp.arange(N, dtype=edge_index.dtype)
    src = jnp.concatenate([edge_index[0], loop])
    dst = jnp.concatenate([edge_index[1], loop])
    e0 = src.shape[0]
    srcp = jnp.zeros((E_PAD,), jnp.int32).at[:e0].set(src)
    dstp = jnp.full((E_PAD,), N, jnp.int32).at[:e0].set(dst)
    srcs = srcp.reshape(NW, NCH, K)
    dsts = dstp.reshape(NW, NCH, K)

    x_all = jnp.zeros((T, N_PAD, IN_CH), f32).at[:, :N].set(x_seq)

    xwc1_all, ed1_all = _tc_pre(x_all.reshape(T * N_PAD, IN_CH), wcomb1, wd1)
    xwc1_all = xwc1_all.reshape(T, N_PAD, R1)
    ed1_all = ed1_all.reshape(T, N_PAD, 16)

    sc_agg1 = _make_sc_edge_agg(R1, 4, SB1, EB1)
    sc_agg2 = _make_sc_edge_agg(R2, 1, SB2, EB2)
    tok = jnp.zeros((16,), f32)
    parts1 = []
    for t in range(T):
        p = sc_agg1(xwc1_all[t], ed1_all[t], srcs, dsts, tok)
        parts1.append(p)
        tok = p[0, :16, 0]

    h0 = jnp.zeros((N_PAD, GRU_H), f32)
    h1 = jnp.zeros((N_PAD, GRU_H), f32)
    for t in range(T):
        xn, xwc2, ed2 = _tc_mid(parts1[t], x_all[t], b1r, bsel1, wcomb2, wd2)
        part2 = sc_agg2(xwc2, ed2, srcs, dsts, tok)
        tok = part2[0, :16, 0]
        h0, h1 = _tc_post(part2, xn, h0, h1, b2r, bsel2, P2,
                          wih0t, whh0t, bih0r, bhh0r,
                          wih1t, whh1t, bih1r, bhh1r)

    out = _tc_head(h1, wfht, bfhr, wfo128, bfo128)
    return out[:N, 0:1]


# speedup vs baseline: 26.1230x; 1.3586x over previous
import functools

import jax
import jax.numpy as jnp
from jax import lax
from jax.experimental import pallas as pl
from jax.experimental.pallas import tpu as pltpu
from jax.experimental.pallas import tpu_sc as plsc

N = 10000
N_PAD = 10240
T = 4
IN_CH = 128
GAT_C = 32
GRU_H = 32

NW = 32
EPW = 5376
E_PAD = NW * EPW

R1 = 144
SB1, EB1 = 128, 136
R2 = 64
SB2, EB2 = 32, 40

ROWS_PER_TILE = N_PAD // 16


@functools.lru_cache(maxsize=None)
def _make_sc_edge_agg(R, H, SB, EB, K, RING):
    NCH = EPW // K
    ZCP = ROWS_PER_TILE // K
    mesh = plsc.VectorSubcoreMesh(core_axis_name="c", subcore_axis_name="s")

    scratch = [
        pltpu.VMEM((NCH, K), jnp.int32),
        pltpu.VMEM((NCH, K), jnp.int32),
    ]
    scratch += [pltpu.VMEM((K, R), jnp.float32) for _ in range(RING)]
    scratch += [pltpu.VMEM((K, 16), jnp.float32) for _ in range(RING)]
    scratch += [pltpu.VMEM_SHARED((N_PAD, R), jnp.float32)]
    scratch += [pltpu.SemaphoreType.DMA for _ in range(2 * RING)]

    @functools.partial(
        pl.kernel,
        out_type=jax.ShapeDtypeStruct((2, N_PAD, R), jnp.float32),
        mesh=mesh,
        compiler_params=pltpu.CompilerParams(
            use_tc_tiling_on_sc=False, needs_layout_passes=False),
        scratch_types=scratch,
    )
    def sc_kernel(xwc_hbm, ed_hbm, src_hbm, dst_hbm, tok_hbm, out_hbm, *scr):
        del tok_hbm
        src_w, dst_w = scr[0], scr[1]
        rows_ring = scr[2:2 + RING]
        de_ring = scr[2 + RING:2 + 2 * RING]
        acc = scr[2 + 2 * RING]
        gsems = scr[3 + 2 * RING:3 + 3 * RING]
        ssems = scr[3 + 3 * RING:3 + 4 * RING]
        bufs = tuple(zip(rows_ring, de_ring, gsems, ssems))
        rows = rows_ring[0]
        cid = lax.axis_index("c")
        sid = lax.axis_index("s")
        wid = sid * 2 + cid
        lanes0 = lax.iota(jnp.int32, 16)
        zv = jnp.zeros((16,), jnp.float32)

        pltpu.sync_copy(src_hbm.at[wid], src_w)
        pltpu.sync_copy(dst_hbm.at[wid], dst_w)

        def zrow(r, c):
            def zcol(k, c2):
                plsc.store_scatter(
                    rows, [jnp.full((16,), r, jnp.int32), k * 16 + lanes0], zv)
                return c2
            return lax.fori_loop(0, R // 16, zcol, c)
        lax.fori_loop(0, K, zrow, 0)

        base_n = sid * ROWS_PER_TILE

        def zcp(i, c):
            pltpu.sync_copy(rows, acc.at[pl.ds(base_n + i * K, K)])
            return c
        lax.fori_loop(0, ZCP, zcp, 0)
        plsc.subcore_barrier()

        def fire_gather(j, b):
            rb, db, gsem, _ = bufs[b]
            pltpu.async_copy(xwc_hbm.at[src_w.at[j]], rb, gsem)
            pltpu.async_copy(ed_hbm.at[dst_w.at[j]], db, gsem)

        def wait_gather(j, b):
            rb, db, gsem, _ = bufs[b]
            pltpu.make_async_copy(xwc_hbm.at[src_w.at[j]], rb, gsem).wait()
            pltpu.make_async_copy(ed_hbm.at[dst_w.at[j]], db, gsem).wait()

        def fire_scatter(j, b):
            rb, _, _, ssem = bufs[b]
            pltpu.async_copy(rb, acc.at[dst_w.at[j]], ssem, add=True)

        def wait_scatter(j, b):
            rb, _, _, ssem = bufs[b]
            pltpu.make_async_copy(rb, acc.at[dst_w.at[j]], ssem).wait()

        def compute(b):
            rb, db, _, _ = bufs[b]

            def grp(g, c2):
                lanes = g * 16 + lanes0
                for h in range(H):
                    s = plsc.load_gather(
                        rb, [lanes, jnp.full((16,), SB + h, jnp.int32)])
                    d = plsc.load_gather(
                        db, [lanes, jnp.full((16,), h, jnp.int32)])
                    a = s + d
                    a = jnp.maximum(a, 0.2 * a)
                    e = jnp.exp(a)
                    plsc.store_scatter(
                        rb, [lanes, jnp.full((16,), EB + h, jnp.int32)], e)

                    def col(cc, c3):
                        colv = jnp.full((16,), 0, jnp.int32) + cc
                        v = plsc.load_gather(rb, [lanes, colv]) * e
                        plsc.store_scatter(rb, [lanes, colv], v)
                        return c3
                    lax.fori_loop(h * 32, h * 32 + 32, col, c2)
                return c2
            lax.fori_loop(0, K // 16, grp, 0)

        fire_gather(0, 0)

        def group(g, c):
            for p in range(RING):
                jp = RING * g + p
                jn = jp + 1
                nb = (p + 1) % RING
                if p == RING - 1:
                    @pl.when(jn < NCH)
                    def _():
                        wait_scatter(jp, nb)
                        fire_gather(jn, nb)
                else:
                    @pl.when(jn >= RING)
                    def _():
                        wait_scatter(jp, nb)
                    fire_gather(jn, nb)
                wait_gather(jp, p)
                compute(p)
                fire_scatter(jp, p)
            return c
        lax.fori_loop(0, NCH // RING, group, 0)
        for i in range(RING):
            wait_scatter(NCH - RING + i, i)
        plsc.subcore_barrier()

        def wout(i, c):
            pltpu.sync_copy(acc.at[pl.ds(base_n + i * K, K)],
                            out_hbm.at[cid, pl.ds(base_n + i * K, K)])
            return c
        lax.fori_loop(0, ZCP, wout, 0)

    return sc_kernel


_BLK = 512


def _tc_pre_body(x_ref, wc_ref, wd_ref, xwc_ref, ed_ref):
    x = x_ref[...]
    xwc_ref[...] = jnp.dot(x, wc_ref[...], preferred_element_type=jnp.float32)
    ed_ref[...] = jnp.dot(x, wd_ref[...], preferred_element_type=jnp.float32)


def _tc_pre(x_all, wcomb1, wd1):
    g = x_all.shape[0] // _BLK
    return pl.pallas_call(
        _tc_pre_body,
        grid=(g,),
        in_specs=[
            pl.BlockSpec((_BLK, IN_CH), lambda i: (i, 0)),
            pl.BlockSpec((IN_CH, R1), lambda i: (0, 0)),
            pl.BlockSpec((IN_CH, 16), lambda i: (0, 0)),
        ],
        out_specs=[
            pl.BlockSpec((_BLK, R1), lambda i: (i, 0)),
            pl.BlockSpec((_BLK, 16), lambda i: (i, 0)),
        ],
        out_shape=[
            jax.ShapeDtypeStruct((x_all.shape[0], R1), jnp.float32),
            jax.ShapeDtypeStruct((x_all.shape[0], 16), jnp.float32),
        ],
    )(x_all, wcomb1, wd1)


def _tc_mid_body(p_ref, x_ref, b1_ref, bsel_ref, wc2_ref, wd2_ref,
                 xn_ref, xwc2_ref, ed2_ref):
    num = p_ref[0, :, :IN_CH] + p_ref[1, :, :IN_CH]
    tail = p_ref[0, :, IN_CH:R1] + p_ref[1, :, IN_CH:R1]
    den = jnp.dot(tail, bsel_ref[...], preferred_element_type=jnp.float32)
    g1 = num / (den + 1e-16) + b1_ref[...]
    xn = jnp.where(g1 > 0, g1, jnp.exp(g1) - 1.0) + x_ref[...]
    xn_ref[...] = xn
    xwc2_ref[...] = jnp.dot(xn, wc2_ref[...], preferred_element_type=jnp.float32)
    ed2_ref[...] = jnp.dot(xn, wd2_ref[...], preferred_element_type=jnp.float32)


def _tc_mid(part1, x_t, b1r, bsel1, wcomb2, wd2):
    g = N_PAD // _BLK
    return pl.pallas_call(
        _tc_mid_body,
        grid=(g,),
        in_specs=[
            pl.BlockSpec((2, _BLK, R1), lambda i: (0, i, 0)),
            pl.BlockSpec((_BLK, IN_CH), lambda i: (i, 0)),
            pl.BlockSpec((1, IN_CH), lambda i: (0, 0)),
            pl.BlockSpec((R1 - IN_CH, IN_CH), lambda i: (0, 0)),
            pl.BlockSpec((IN_CH, R2), lambda i: (0, 0)),
            pl.BlockSpec((IN_CH, 16), lambda i: (0, 0)),
        ],
        out_specs=[
            pl.BlockSpec((_BLK, IN_CH), lambda i: (i, 0)),
            pl.BlockSpec((_BLK, R2), lambda i: (i, 0)),
            pl.BlockSpec((_BLK, 16), lambda i: (i, 0)),
        ],
        out_shape=[
            jax.ShapeDtypeStruct((N_PAD, IN_CH), jnp.float32),
            jax.ShapeDtypeStruct((N_PAD, R2), jnp.float32),
            jax.ShapeDtypeStruct((N_PAD, 16), jnp.float32),
        ],
    )(part1, x_t, b1r, bsel1, wcomb2, wd2)


def _gru_block(x, h, wih_t, whh_t, bih, bhh):
    gi = jnp.dot(x, wih_t, preferred_element_type=jnp.float32) + bih
    gh = jnp.dot(h, whh_t, preferred_element_type=jnp.float32) + bhh
    r = jax.nn.sigmoid(gi[:, 0:32] + gh[:, 0:32])
    z = jax.nn.sigmoid(gi[:, 32:64] + gh[:, 32:64])
    ng = jnp.tanh(gi[:, 64:96] + r * gh[:, 64:96])
    return (1.0 - z) * ng + z * h


def _tc_post_body(p_ref, xn_ref, h0_ref, h1_ref, b2_ref, bsel2_ref, p2_ref,
                  wih0_ref, whh0_ref, bih0_ref, bhh0_ref,
                  wih1_ref, whh1_ref, bih1_ref, bhh1_ref,
                  h0n_ref, h1n_ref):
    num = p_ref[0, :, :GAT_C] + p_ref[1, :, :GAT_C]
    tail = p_ref[0, :, GAT_C:R2] + p_ref[1, :, GAT_C:R2]
    den = jnp.dot(tail, bsel2_ref[...], preferred_element_type=jnp.float32)
    g2 = num / (den + 1e-16) + b2_ref[...]
    x2 = (jnp.where(g2 > 0, g2, jnp.exp(g2) - 1.0)
          + jnp.dot(xn_ref[...], p2_ref[...], preferred_element_type=jnp.float32))
    h0 = h0_ref[...]
    h1 = h1_ref[...]
    h0n = _gru_block(x2, h0, wih0_ref[...], whh0_ref[...],
                     bih0_ref[...], bhh0_ref[...])
    h1n = _gru_block(h0n, h1, wih1_ref[...], whh1_ref[...],
                     bih1_ref[...], bhh1_ref[...])
    h0n_ref[...] = h0n
    h1n_ref[...] = h1n


def _tc_post(part2, xn, h0, h1, b2r, bsel2, p2,
             wih0t, whh0t, bih0r, bhh0r, wih1t, whh1t, bih1r, bhh1r):
    g = N_PAD // _BLK
    full = lambda a, b: pl.BlockSpec((a, b), lambda i: (0, 0))
    rows = lambda b: pl.BlockSpec((_BLK, b), lambda i: (i, 0))
    return pl.pallas_call(
        _tc_post_body,
        grid=(g,),
        in_specs=[
            pl.BlockSpec((2, _BLK, R2), lambda i: (0, i, 0)),
            rows(IN_CH), rows(GRU_H), rows(GRU_H),
            full(1, GAT_C), full(R2 - GAT_C, GAT_C), full(IN_CH, GAT_C),
            full(GAT_C, 96), full(GRU_H, 96), full(1, 96), full(1, 96),
            full(GRU_H, 96), full(GRU_H, 96), full(1, 96), full(1, 96),
        ],
        out_specs=[rows(GRU_H), rows(GRU_H)],
        out_shape=[
            jax.ShapeDtypeStruct((N_PAD, GRU_H), jnp.float32),
            jax.ShapeDtypeStruct((N_PAD, GRU_H), jnp.float32),
        ],
    )(part2, xn, h0, h1, b2r, bsel2, p2,
      wih0t, whh0t, bih0r, bhh0r, wih1t, whh1t, bih1r, bhh1r)


def _tc_head_body(h1_ref, wfh_ref, bfh_ref, wfo_ref, bfo_ref, out_ref):
    y = jnp.dot(h1_ref[...], wfh_ref[...], preferred_element_type=jnp.float32)
    y = jnp.maximum(y + bfh_ref[...], 0.0)
    out_ref[...] = (jnp.dot(y, wfo_ref[...], preferred_element_type=jnp.float32)
                    + bfo_ref[...])


def _tc_head(h1, wfht, bfhr, wfo128, bfo128):
    g = N_PAD // _BLK
    full = lambda a, b: pl.BlockSpec((a, b), lambda i: (0, 0))
    return pl.pallas_call(
        _tc_head_body,
        grid=(g,),
        in_specs=[
            pl.BlockSpec((_BLK, GRU_H), lambda i: (i, 0)),
            full(GRU_H, 32), full(1, 32), full(32, 128), full(1, 128),
        ],
        out_specs=pl.BlockSpec((_BLK, 128), lambda i: (i, 0)),
        out_shape=jax.ShapeDtypeStruct((N_PAD, 128), jnp.float32),
    )(h1, wfht, bfhr, wfo128, bfo128)


def kernel(x_seq, edge_index, W1, a_src1, a_dst1, b1, W2, a_src2, a_dst2, b2,
           P2, Wih0, Whh0, bih0, bhh0, Wih1, Whh1, bih1, bhh1,
           Wfh, bfh, Wfo, bfo):
    f32 = jnp.float32

    sel_s1 = jnp.zeros((IN_CH, R1 - IN_CH), f32)
    sel_d1 = jnp.zeros((IN_CH, 16), f32)
    for h in range(4):
        blk = jnp.zeros((IN_CH,), f32).at[32 * h:32 * h + 32].set(a_src1[h])
        sel_s1 = sel_s1.at[:, h].set(blk)
        blkd = jnp.zeros((IN_CH,), f32).at[32 * h:32 * h + 32].set(a_dst1[h])
        sel_d1 = sel_d1.at[:, h].set(blkd)
    wcomb1 = jnp.concatenate([W1, W1 @ sel_s1], axis=1)
    wd1 = W1 @ sel_d1

    sel_s2 = jnp.zeros((GAT_C, R2 - GAT_C), f32).at[:, 0].set(a_src2[0])
    sel_d2 = jnp.zeros((GAT_C, 16), f32).at[:, 0].set(a_dst2[0])
    wcomb2 = jnp.concatenate([W2, W2 @ sel_s2], axis=1)
    wd2 = W2 @ sel_d2

    bsel1 = jnp.zeros((R1 - IN_CH, IN_CH), f32)
    for h in range(4):
        bsel1 = bsel1.at[EB1 - IN_CH + h, 32 * h:32 * h + 32].set(1.0)
    bsel2 = jnp.zeros((R2 - GAT_C, GAT_C), f32).at[EB2 - GAT_C, :].set(1.0)

    def pad_cols(m, w):
        return jnp.zeros((m.shape[0], w), f32).at[:, :m.shape[1]].set(m)

    wih0t, whh0t = Wih0.T, Whh0.T
    wih1t, whh1t = Wih1.T, Whh1.T
    bih0r, bhh0r = bih0.reshape(1, 96), bhh0.reshape(1, 96)
    bih1r, bhh1r = bih1.reshape(1, 96), bhh1.reshape(1, 96)
    b1r = b1.reshape(1, IN_CH)
    b2r = b2.reshape(1, GAT_C)
    wfht = Wfh.T
    bfhr = bfh.reshape(1, 32)
    wfo128 = pad_cols(Wfo.T, 128)
    bfo128 = pad_cols(bfo.reshape(1, 1), 128)

    loop = jnp.arange(N, dtype=edge_index.dtype)
    src = jnp.concatenate([edge_index[0], loop])
    dst = jnp.concatenate([edge_index[1], loop])
    e0 = src.shape[0]
    srcp = jnp.zeros((E_PAD,), jnp.int32).at[:e0].set(src)
    dstp = jnp.full((E_PAD,), N, jnp.int32).at[:e0].set(dst)
    srcs1 = srcp.reshape(NW, EPW // 64, 64)
    dsts1 = dstp.reshape(NW, EPW // 64, 64)
    srcs2 = srcp.reshape(NW, EPW // 128, 128)
    dsts2 = dstp.reshape(NW, EPW // 128, 128)

    x_all = jnp.zeros((T, N_PAD, IN_CH), f32).at[:, :N].set(x_seq)

    xwc1_all, ed1_all = _tc_pre(x_all.reshape(T * N_PAD, IN_CH), wcomb1, wd1)
    xwc1_all = xwc1_all.reshape(T, N_PAD, R1)
    ed1_all = ed1_all.reshape(T, N_PAD, 16)

    sc_agg1 = _make_sc_edge_agg(R1, 4, SB1, EB1, 64, 2)
    sc_agg2 = _make_sc_edge_agg(R2, 1, SB2, EB2, 128, 3)
    tok = jnp.zeros((16,), f32)
    parts1 = []
    for t in range(T):
        p = sc_agg1(xwc1_all[t], ed1_all[t], srcs1, dsts1, tok)
        parts1.append(p)
        tok = p[0, :16, 0]

    h0 = jnp.zeros((N_PAD, GRU_H), f32)
    h1 = jnp.zeros((N_PAD, GRU_H), f32)
    for t in range(T):
        xn, xwc2, ed2 = _tc_mid(parts1[t], x_all[t], b1r, bsel1, wcomb2, wd2)
        part2 = sc_agg2(xwc2, ed2, srcs2, dsts2, tok)
        tok = part2[0, :16, 0]
        h0, h1 = _tc_post(part2, xn, h0, h1, b2r, bsel2, P2,
                          wih0t, whh0t, bih0r, bhh0r,
                          wih1t, whh1t, bih1r, bhh1r)

    out = _tc_head(h1, wfht, bfhr, wfo128, bfo128)
    return out[:N, 0:1]
